# Initial kernel scaffold; baseline (speedup 1.0000x reference)
import jax
import jax.numpy as jnp
from jax.experimental import pallas as pl


def kernel(x, batch):
    raise NotImplementedError("write your pallas kernel here")


# speedup vs baseline: 3.8957x; 3.8957x over previous
import functools

import jax
import jax.numpy as jnp
from jax.experimental import pallas as pl
from jax.experimental.pallas import tpu as pltpu

_B = 512
_K = 3
_R = 2048
_BIG = 1 << 30
_INTERPRET = False


def _pool_kernel(xref, bref, kref, sums_ref, mean_ref, sel_ref, cnt_ref,
                 *, nblocks, n):
    g = pl.program_id(0)

    @pl.when(g == 0)
    def _init():
        sums_ref[...] = jnp.zeros_like(sums_ref)
        cnt_ref[...] = jnp.zeros_like(cnt_ref)

    xb = xref[...]
    rowid = g * _R + jax.lax.broadcasted_iota(jnp.int32, (_R, 1), 0)
    xb = jnp.where(rowid < n, xb, 0.0)
    bt = bref[g]
    seg = jax.lax.broadcasted_iota(jnp.int32, (_B, 1), 0)
    present = bt == seg
    sums_ref[...] += jnp.dot(present.astype(jnp.bfloat16),
                             xb.astype(jnp.bfloat16),
                             preferred_element_type=jnp.float32)
    cnt_ref[:, 0:1] = cnt_ref[:, 0:1] + jnp.sum(
        present.astype(jnp.float32), axis=1, keepdims=True)

    @pl.when(g == nblocks - 1)
    def _finalize():
        cnt = cnt_ref[:, 0:1]
        mean_ref[...] = sums_ref[...] / jnp.maximum(cnt, 1.0)

        neg = jnp.float32(-jnp.inf)

        def run_pass(pk, pi):
            def body(j, carry):
                mk, mi = carry
                kj = kref[j]
                bj = bref[j]
                idx = j * _R + jax.lax.broadcasted_iota(jnp.int32, (1, _R), 1)
                cand = (bj == seg) & ((kj < pk) | ((kj == pk) & (idx > pi)))
                ck = jnp.where(cand, kj, neg)
                pm = jnp.max(ck, axis=1, keepdims=True)
                ci = jnp.where(cand & (kj == pm), idx, _BIG)
                pmin = jnp.min(ci, axis=1, keepdims=True)
                take = (pm > mk) | ((pm == mk) & (pmin < mi))
                return (jnp.where(take, pm, mk), jnp.where(take, pmin, mi))

            mk0 = jnp.full((_B, 1), neg, jnp.float32)
            mi0 = jnp.full((_B, 1), _BIG, jnp.int32)
            return jax.lax.fori_loop(0, nblocks, body, (mk0, mi0))

        k1, i1 = run_pass(jnp.full((_B, 1), jnp.inf, jnp.float32),
                          jnp.full((_B, 1), -1, jnp.int32))
        k2, i2 = run_pass(k1, i1)
        k3, i3 = run_pass(k2, i2)
        s1 = jnp.where(k1 > neg, i1, -1)
        s2 = jnp.where(k2 > neg, i2, -1)
        s3 = jnp.where(k3 > neg, i3, -1)
        lane = jax.lax.broadcasted_iota(jnp.int32, (_B, 128), 1)
        sel = jnp.where(lane == 0, s1,
                        jnp.where(lane == 1, s2,
                                  jnp.where(lane == 2, s3, -1)))
        sel_ref[...] = sel


def _gather_kernel(xref, sref, out_ref, *, n):
    g = pl.program_id(0)

    @pl.when(g == 0)
    def _init():
        out_ref[...] = jnp.zeros_like(out_ref)

    xb = xref[...]
    rowid = g * _R + jax.lax.broadcasted_iota(jnp.int32, (_R, 1), 0)
    xb = jnp.where(rowid < n, xb, 0.0)
    sb = sref[...]
    idx = g * _R + jax.lax.broadcasted_iota(jnp.int32, (1, _R), 1)
    smat = (sb == idx).astype(jnp.bfloat16)
    out_ref[...] += jnp.dot(smat, xb.astype(jnp.bfloat16),
                            preferred_element_type=jnp.float32)


def kernel(x, batch):
    n, d = x.shape
    nb = pl.cdiv(n, _R)
    npad = nb * _R
    batchp = jnp.pad(batch.astype(jnp.int32), (0, npad - n),
                     constant_values=_B).reshape(nb, 1, _R)
    keyp = jnp.pad(x[:, d - 1], (0, npad - n),
                   constant_values=-jnp.inf).reshape(nb, 1, _R)

    sums, mean, sel = pl.pallas_call(
        functools.partial(_pool_kernel, nblocks=nb, n=n),
        grid=(nb,),
        in_specs=[
            pl.BlockSpec((_R, d), lambda g: (g, 0)),
            pl.BlockSpec((nb, 1, _R), lambda g: (0, 0, 0)),
            pl.BlockSpec((nb, 1, _R), lambda g: (0, 0, 0)),
        ],
        out_specs=[
            pl.BlockSpec((_B, d), lambda g: (0, 0)),
            pl.BlockSpec((_B, d), lambda g: (0, 0)),
            pl.BlockSpec((_B, 128), lambda g: (0, 0)),
        ],
        out_shape=[
            jax.ShapeDtypeStruct((_B, d), jnp.float32),
            jax.ShapeDtypeStruct((_B, d), jnp.float32),
            jax.ShapeDtypeStruct((_B, 128), jnp.int32),
        ],
        scratch_shapes=[pltpu.VMEM((_B, 128), jnp.float32)],
        interpret=_INTERPRET,
    )(x, batchp, keyp)

    sel3 = sel[:, :_K].reshape(_B * _K, 1)
    rows = pl.pallas_call(
        functools.partial(_gather_kernel, n=n),
        grid=(nb,),
        in_specs=[
            pl.BlockSpec((_R, d), lambda g: (g, 0)),
            pl.BlockSpec((_B * _K, 1), lambda g: (0, 0)),
        ],
        out_specs=pl.BlockSpec((_B * _K, d), lambda g: (0, 0)),
        out_shape=jax.ShapeDtypeStruct((_B * _K, d), jnp.float32),
        interpret=_INTERPRET,
    )(x, sel3)

    topk = rows.reshape(_B, _K * d)
    return jnp.concatenate([mean, sums, topk], axis=-1)


# speedup vs baseline: 8.6877x; 2.2301x over previous
import functools

import jax
import jax.numpy as jnp
from jax.experimental import pallas as pl
from jax.experimental.pallas import tpu as pltpu

_B = 512
_K = 3
_R = 2048
_W = 32
_NEG = float("-inf")
_BIGF = 1e9
_INTERPRET = False


def _pool_kernel(xref, bref, kref, sums_ref, mean_ref, sel_ref, cnt_ref,
                 top3_ref, *, nblocks, n):
    g = pl.program_id(0)

    @pl.when(g == 0)
    def _init():
        sums_ref[...] = jnp.zeros_like(sums_ref)
        cnt_ref[...] = jnp.zeros_like(cnt_ref)
        lane = jax.lax.broadcasted_iota(jnp.int32, (_B, 128), 1)
        top3_ref[...] = jnp.where(lane < 3, _NEG, _BIGF)

    xb = xref[...]
    rowid = g * _R + jax.lax.broadcasted_iota(jnp.int32, (_R, 1), 0)
    xb = jnp.where(rowid < n, xb, 0.0)
    bt = bref[g]
    seg = jax.lax.broadcasted_iota(jnp.int32, (_B, 1), 0)
    pb = (bt == seg).astype(jnp.bfloat16)
    sums_ref[...] += jnp.dot(pb, xb.astype(jnp.bfloat16),
                             preferred_element_type=jnp.float32)
    cnt_ref[...] += jnp.dot(pb, jnp.ones((_R, 128), jnp.bfloat16),
                            preferred_element_type=jnp.float32)

    kj = kref[g]
    idxf = (g * _R + jax.lax.broadcasted_iota(jnp.int32, (1, _R), 1)
            ).astype(jnp.float32)
    w0 = bt[0, 0] // _W
    w1 = jnp.minimum(bt[0, _R - 1], _B - 1) // _W
    wseg = jax.lax.broadcasted_iota(jnp.int32, (_W, 1), 0)
    lane8 = jax.lax.broadcasted_iota(jnp.int32, (_W, 8), 1)

    def wbody(w, carry):
        act = bt == (w * _W + wseg)
        loc = []
        for _ in range(_K):
            ck = jnp.where(act, kj, _NEG)
            m = jnp.max(ck, axis=1, keepdims=True)
            eqm = act & (kj == m)
            im = jnp.min(jnp.where(eqm, idxf, _BIGF), axis=1, keepdims=True)
            act = act & ~(eqm & (idxf == im))
            loc.append((m, im))
        prev = top3_ref[pl.ds(w * _W, _W), :]
        ck8 = jnp.full((_W, 8), _NEG, jnp.float32)
        ci8 = jnp.full((_W, 8), _BIGF, jnp.float32)
        for t in range(_K):
            ck8 = jnp.where(lane8 == t, loc[t][0], ck8)
            ci8 = jnp.where(lane8 == t, loc[t][1], ci8)
            ck8 = jnp.where(lane8 == _K + t, prev[:, t:t + 1], ck8)
            ci8 = jnp.where(lane8 == _K + t, prev[:, _K + t:_K + t + 1], ci8)
        out = jnp.full((_W, 128), _BIGF, jnp.float32)
        lanew = jax.lax.broadcasted_iota(jnp.int32, (_W, 128), 1)
        for t in range(_K):
            mk = jnp.max(ck8, axis=1, keepdims=True)
            mi = jnp.min(jnp.where(ck8 == mk, ci8, _BIGF), axis=1,
                         keepdims=True)
            kill = (ck8 == mk) & (ci8 == mi)
            ck8 = jnp.where(kill, _NEG, ck8)
            ci8 = jnp.where(kill, _BIGF, ci8)
            out = jnp.where(lanew == t, mk, out)
            out = jnp.where(lanew == _K + t, mi, out)
        top3_ref[pl.ds(w * _W, _W), :] = out
        return carry

    jax.lax.fori_loop(w0, w1 + 1, wbody, jnp.int32(0))

    @pl.when(g == nblocks - 1)
    def _finalize():
        cnt = cnt_ref[:, 0:1]
        mean_ref[...] = sums_ref[...] / jnp.maximum(cnt, 1.0)
        t3 = top3_ref[...]
        lane = jax.lax.broadcasted_iota(jnp.int32, (_B, 128), 1)
        sel = jnp.full((_B, 128), -1, jnp.int32)
        for t in range(_K):
            st = jnp.where(t3[:, t:t + 1] > _NEG,
                           t3[:, _K + t:_K + t + 1].astype(jnp.int32), -1)
            sel = jnp.where(lane == t, st, sel)
        sel_ref[...] = sel


def _gather_kernel(xref, bref, sref, out_ref, *, n):
    g = pl.program_id(0)

    @pl.when(g == 0)
    def _init():
        out_ref[...] = jnp.zeros_like(out_ref)

    xb = xref[...]
    rowid = g * _R + jax.lax.broadcasted_iota(jnp.int32, (_R, 1), 0)
    xb = jnp.where(rowid < n, xb, 0.0).astype(jnp.bfloat16)
    bt = bref[g]
    idx = g * _R + jax.lax.broadcasted_iota(jnp.int32, (1, _R), 1)
    w0 = bt[0, 0] // _W
    w1 = jnp.minimum(bt[0, _R - 1], _B - 1) // _W

    def wbody(w, carry):
        selw = sref[pl.ds(w * _W * _K, _W * _K), :]
        smat = (selw == idx).astype(jnp.bfloat16)
        out_ref[pl.ds(w * _W * _K, _W * _K), :] += jnp.dot(
            smat, xb, preferred_element_type=jnp.float32)
        return carry

    jax.lax.fori_loop(w0, w1 + 1, wbody, jnp.int32(0))


def kernel(x, batch):
    n, d = x.shape
    nb = pl.cdiv(n, _R)
    npad = nb * _R
    batchp = jnp.pad(batch.astype(jnp.int32), (0, npad - n),
                     constant_values=_B).reshape(nb, 1, _R)
    keyp = jnp.pad(x[:, d - 1], (0, npad - n),
                   constant_values=_NEG).reshape(nb, 1, _R)

    sums, mean, sel = pl.pallas_call(
        functools.partial(_pool_kernel, nblocks=nb, n=n),
        grid=(nb,),
        in_specs=[
            pl.BlockSpec((_R, d), lambda g: (g, 0)),
            pl.BlockSpec((nb, 1, _R), lambda g: (0, 0, 0)),
            pl.BlockSpec((nb, 1, _R), lambda g: (0, 0, 0)),
        ],
        out_specs=[
            pl.BlockSpec((_B, d), lambda g: (0, 0)),
            pl.BlockSpec((_B, d), lambda g: (0, 0)),
            pl.BlockSpec((_B, 128), lambda g: (0, 0)),
        ],
        out_shape=[
            jax.ShapeDtypeStruct((_B, d), jnp.float32),
            jax.ShapeDtypeStruct((_B, d), jnp.float32),
            jax.ShapeDtypeStruct((_B, 128), jnp.int32),
        ],
        scratch_shapes=[pltpu.VMEM((_B, 128), jnp.float32),
                        pltpu.VMEM((_B, 128), jnp.float32)],
        interpret=_INTERPRET,
    )(x, batchp, keyp)

    sel3 = sel[:, :_K].reshape(_B * _K, 1)
    rows = pl.pallas_call(
        functools.partial(_gather_kernel, n=n),
        grid=(nb,),
        in_specs=[
            pl.BlockSpec((_R, d), lambda g: (g, 0)),
            pl.BlockSpec((nb, 1, _R), lambda g: (0, 0, 0)),
            pl.BlockSpec((_B * _K, 1), lambda g: (0, 0)),
        ],
        out_specs=pl.BlockSpec((_B * _K, d), lambda g: (0, 0)),
        out_shape=jax.ShapeDtypeStruct((_B * _K, d), jnp.float32),
        interpret=_INTERPRET,
    )(x, batchp, sel3)

    topk = rows.reshape(_B, _K * d)
    return jnp.concatenate([mean, sums, topk], axis=-1)


# speedup vs baseline: 11.6250x; 1.3381x over previous
import functools

import jax
import jax.numpy as jnp
from jax.experimental import pallas as pl
from jax.experimental.pallas import tpu as pltpu

_B = 512
_K = 3
_R = 2048
_W = 32
_NEG = float("-inf")
_BIGF = 1e9
_INTERPRET = False


def _pool_kernel(xref, bref, sums_ref, mean_ref, sel_ref, cnt_ref,
                 top3_ref, *, nblocks, n):
    g = pl.program_id(0)

    @pl.when(g == 0)
    def _init():
        sums_ref[...] = jnp.zeros_like(sums_ref)
        cnt_ref[...] = jnp.zeros_like(cnt_ref)
        lane = jax.lax.broadcasted_iota(jnp.int32, (_B, 128), 1)
        top3_ref[...] = jnp.where(lane < 3, _NEG, _BIGF)

    xb = xref[...]
    rowid = g * _R + jax.lax.broadcasted_iota(jnp.int32, (_R, 1), 0)
    xb = jnp.where(rowid < n, xb, 0.0)
    xbf = xb.astype(jnp.bfloat16)
    bt = bref[g]

    kj = jnp.transpose(xb[:, -1:])
    idxf = (g * _R + jax.lax.broadcasted_iota(jnp.int32, (1, _R), 1)
            ).astype(jnp.float32)
    w0 = bt[0, 0] // _W
    w1 = jnp.minimum(bt[0, _R - 1], _B - 1) // _W
    wseg = jax.lax.broadcasted_iota(jnp.int32, (_W, 1), 0)
    lane8 = jax.lax.broadcasted_iota(jnp.int32, (_W, 8), 1)

    def wbody(w, carry):
        act = bt == (w * _W + wseg)
        sums_ref[pl.ds(w * _W, _W), :] += jnp.dot(
            act.astype(jnp.bfloat16), xbf,
            preferred_element_type=jnp.float32)
        cnt_ref[pl.ds(w * _W, _W), 0:1] += jnp.sum(
            act.astype(jnp.float32), axis=1, keepdims=True)
        loc = []
        for _ in range(_K):
            ck = jnp.where(act, kj, _NEG)
            m = jnp.max(ck, axis=1, keepdims=True)
            eqm = act & (kj == m)
            im = jnp.min(jnp.where(eqm, idxf, _BIGF), axis=1, keepdims=True)
            act = act & ~(eqm & (idxf == im))
            loc.append((m, im))
        prev = top3_ref[pl.ds(w * _W, _W), :]
        ck8 = jnp.full((_W, 8), _NEG, jnp.float32)
        ci8 = jnp.full((_W, 8), _BIGF, jnp.float32)
        for t in range(_K):
            ck8 = jnp.where(lane8 == t, loc[t][0], ck8)
            ci8 = jnp.where(lane8 == t, loc[t][1], ci8)
            ck8 = jnp.where(lane8 == _K + t, prev[:, t:t + 1], ck8)
            ci8 = jnp.where(lane8 == _K + t, prev[:, _K + t:_K + t + 1], ci8)
        out = jnp.full((_W, 128), _BIGF, jnp.float32)
        lanew = jax.lax.broadcasted_iota(jnp.int32, (_W, 128), 1)
        for t in range(_K):
            mk = jnp.max(ck8, axis=1, keepdims=True)
            mi = jnp.min(jnp.where(ck8 == mk, ci8, _BIGF), axis=1,
                         keepdims=True)
            kill = (ck8 == mk) & (ci8 == mi)
            ck8 = jnp.where(kill, _NEG, ck8)
            ci8 = jnp.where(kill, _BIGF, ci8)
            out = jnp.where(lanew == t, mk, out)
            out = jnp.where(lanew == _K + t, mi, out)
        top3_ref[pl.ds(w * _W, _W), :] = out
        return carry

    jax.lax.fori_loop(w0, w1 + 1, wbody, jnp.int32(0))

    @pl.when(g == nblocks - 1)
    def _finalize():
        cnt = cnt_ref[:, 0:1]
        mean_ref[...] = sums_ref[...] / jnp.maximum(cnt, 1.0)
        t3 = top3_ref[...]
        lane = jax.lax.broadcasted_iota(jnp.int32, (_B, 128), 1)
        sel = jnp.full((_B, 128), -1, jnp.int32)
        for t in range(_K):
            st = jnp.where(t3[:, t:t + 1] > _NEG,
                           t3[:, _K + t:_K + t + 1].astype(jnp.int32), -1)
            sel = jnp.where(lane == t, st, sel)
        sel_ref[...] = sel


def _gather_kernel(xref, bref, sref, out_ref, *, n):
    g = pl.program_id(0)

    @pl.when(g == 0)
    def _init():
        out_ref[...] = jnp.zeros_like(out_ref)

    xb = xref[...]
    rowid = g * _R + jax.lax.broadcasted_iota(jnp.int32, (_R, 1), 0)
    xb = jnp.where(rowid < n, xb, 0.0).astype(jnp.bfloat16)
    bt = bref[g]
    idx = g * _R + jax.lax.broadcasted_iota(jnp.int32, (1, _R), 1)
    w0 = bt[0, 0] // _W
    w1 = jnp.minimum(bt[0, _R - 1], _B - 1) // _W

    def wbody(w, carry):
        selw = sref[pl.ds(w * _W * _K, _W * _K), :]
        smat = (selw == idx).astype(jnp.bfloat16)
        out_ref[pl.ds(w * _W * _K, _W * _K), :] += jnp.dot(
            smat, xb, preferred_element_type=jnp.float32)
        return carry

    jax.lax.fori_loop(w0, w1 + 1, wbody, jnp.int32(0))


def kernel(x, batch):
    n, d = x.shape
    nb = pl.cdiv(n, _R)
    npad = nb * _R
    batchp = jnp.pad(batch.astype(jnp.int32), (0, npad - n),
                     constant_values=_B).reshape(nb, 1, _R)

    sums, mean, sel = pl.pallas_call(
        functools.partial(_pool_kernel, nblocks=nb, n=n),
        grid=(nb,),
        in_specs=[
            pl.BlockSpec((_R, d), lambda g: (g, 0)),
            pl.BlockSpec((nb, 1, _R), lambda g: (0, 0, 0)),
        ],
        out_specs=[
            pl.BlockSpec((_B, d), lambda g: (0, 0)),
            pl.BlockSpec((_B, d), lambda g: (0, 0)),
            pl.BlockSpec((_B, 128), lambda g: (0, 0)),
        ],
        out_shape=[
            jax.ShapeDtypeStruct((_B, d), jnp.float32),
            jax.ShapeDtypeStruct((_B, d), jnp.float32),
            jax.ShapeDtypeStruct((_B, 128), jnp.int32),
        ],
        scratch_shapes=[pltpu.VMEM((_B, 128), jnp.float32),
                        pltpu.VMEM((_B, 128), jnp.float32)],
        interpret=_INTERPRET,
    )(x, batchp)

    sel3 = sel[:, :_K].reshape(_B * _K, 1)
    rows = pl.pallas_call(
        functools.partial(_gather_kernel, n=n),
        grid=(nb,),
        in_specs=[
            pl.BlockSpec((_R, d), lambda g: (g, 0)),
            pl.BlockSpec((nb, 1, _R), lambda g: (0, 0, 0)),
            pl.BlockSpec((_B * _K, 1), lambda g: (0, 0)),
        ],
        out_specs=pl.BlockSpec((_B * _K, d), lambda g: (0, 0)),
        out_shape=jax.ShapeDtypeStruct((_B * _K, d), jnp.float32),
        interpret=_INTERPRET,
    )(x, batchp, sel3)

    topk = rows.reshape(_B, _K * d)
    return jnp.concatenate([mean, sums, topk], axis=-1)


# speedup vs baseline: 15.3809x; 1.3231x over previous
import functools

import jax
import jax.numpy as jnp
from jax.experimental import pallas as pl
from jax.experimental.pallas import tpu as pltpu

_B = 512
_K = 3
_R = 2048
_W = 32
_NEG = float("-inf")
_BIGF = 1e9
_INTERPRET = False


def _pool_kernel(xref, bref, out_ref, cnt_ref, top3_ref, *, nblocks, n, d):
    g = pl.program_id(0)

    @pl.when(g == 0)
    def _init():
        out_ref[...] = jnp.zeros_like(out_ref)
        cnt_ref[...] = jnp.zeros_like(cnt_ref)
        lane = jax.lax.broadcasted_iota(jnp.int32, (_B, 128), 1)
        top3_ref[...] = jnp.where(lane < _K, _NEG, _BIGF)

    xb = xref[...]
    rowid = g * _R + jax.lax.broadcasted_iota(jnp.int32, (_R, 1), 0)
    xb = jnp.where(rowid < n, xb, 0.0)
    xbf = xb.astype(jnp.bfloat16)
    bt = bref[g]

    kj = jnp.transpose(xb[:, -1:])
    idxf = (g * _R + jax.lax.broadcasted_iota(jnp.int32, (1, _R), 1)
            ).astype(jnp.float32)
    w0 = bt[0, 0] // _W
    w1 = jnp.minimum(bt[0, _R - 1], _B - 1) // _W
    wseg = jax.lax.broadcasted_iota(jnp.int32, (_W, 1), 0)
    lane8 = jax.lax.broadcasted_iota(jnp.int32, (_W, 8), 1)
    blo = jnp.float32(g) * _R

    def wbody(w, carry):
        act = bt == (w * _W + wseg)
        out_ref[pl.ds(w * _W, _W), d:2 * d] += jnp.dot(
            act.astype(jnp.bfloat16), xbf,
            preferred_element_type=jnp.float32)
        cnt_ref[pl.ds(w * _W, _W), 0:1] += jnp.sum(
            act.astype(jnp.float32), axis=1, keepdims=True)

        loc = []
        for _ in range(_K):
            ck = jnp.where(act, kj, _NEG)
            m = jnp.max(ck, axis=1, keepdims=True)
            eqm = act & (kj == m)
            im = jnp.min(jnp.where(eqm, idxf, _BIGF), axis=1, keepdims=True)
            act = act & ~(eqm & (idxf == im))
            loc.append((m, im))

        prev = top3_ref[pl.ds(w * _W, _W), :]
        prevk = [prev[:, t:t + 1] for t in range(_K)]
        previ = [prev[:, _K + t:_K + t + 1] for t in range(_K)]
        ck8 = jnp.full((_W, 8), _NEG, jnp.float32)
        ci8 = jnp.full((_W, 8), _BIGF, jnp.float32)
        for t in range(_K):
            ck8 = jnp.where(lane8 == t, loc[t][0], ck8)
            ci8 = jnp.where(lane8 == t, loc[t][1], ci8)
            ck8 = jnp.where(lane8 == _K + t, prevk[t], ck8)
            ci8 = jnp.where(lane8 == _K + t, previ[t], ci8)
        out3 = jnp.full((_W, 128), _BIGF, jnp.float32)
        lanew = jax.lax.broadcasted_iota(jnp.int32, (_W, 128), 1)
        win = []
        for t in range(_K):
            mk = jnp.max(ck8, axis=1, keepdims=True)
            mi = jnp.min(jnp.where(ck8 == mk, ci8, _BIGF), axis=1,
                         keepdims=True)
            kill = (ck8 == mk) & (ci8 == mi)
            ck8 = jnp.where(kill, _NEG, ck8)
            ci8 = jnp.where(kill, _BIGF, ci8)
            out3 = jnp.where(lanew == t, mk, out3)
            out3 = jnp.where(lanew == _K + t, mi, out3)
            win.append((mk, mi))
        top3_ref[pl.ds(w * _W, _W), :] = out3

        prow = [out_ref[pl.ds(w * _W, _W), 2 * d + t * d:2 * d + (t + 1) * d]
                for t in range(_K)]
        new = []
        for t in range(_K):
            mk, mi = win[t]
            is_loc = mi >= blo
            gmat = ((idxf == mi) & is_loc).astype(jnp.bfloat16)
            acc = jnp.dot(gmat, xbf, preferred_element_type=jnp.float32)
            for s in range(_K):
                f = (~is_loc) & (mk == prevk[s]) & (mi == previ[s])
                acc = acc + f.astype(jnp.float32) * prow[s]
            new.append(acc)
        for t in range(_K):
            out_ref[pl.ds(w * _W, _W), 2 * d + t * d:2 * d + (t + 1) * d] = (
                new[t])
        return carry

    jax.lax.fori_loop(w0, w1 + 1, wbody, jnp.int32(0))

    @pl.when(g == nblocks - 1)
    def _finalize():
        cnt = cnt_ref[:, 0:1]
        out_ref[:, 0:d] = out_ref[:, d:2 * d] / jnp.maximum(cnt, 1.0)


def kernel(x, batch):
    n, d = x.shape
    nb = pl.cdiv(n, _R)
    npad = nb * _R
    batchp = jnp.pad(batch.astype(jnp.int32), (0, npad - n),
                     constant_values=_B).reshape(nb, 1, _R)

    out = pl.pallas_call(
        functools.partial(_pool_kernel, nblocks=nb, n=n, d=d),
        grid=(nb,),
        in_specs=[
            pl.BlockSpec((_R, d), lambda g: (g, 0)),
            pl.BlockSpec((nb, 1, _R), lambda g: (0, 0, 0)),
        ],
        out_specs=pl.BlockSpec((_B, (2 + _K) * d), lambda g: (0, 0)),
        out_shape=jax.ShapeDtypeStruct((_B, (2 + _K) * d), jnp.float32),
        scratch_shapes=[pltpu.VMEM((_B, 128), jnp.float32),
                        pltpu.VMEM((_B, 128), jnp.float32)],
        interpret=_INTERPRET,
    )(x, batchp)
    return out
